# baseline (speedup 1.0000x reference)
import functools
import jax
import jax.numpy as jnp
import numpy as np
from jax.experimental import pallas as pl

NODE_DIM = 128
EDGE_DIM = 16
HIDDEN = 256
HEADS = 8
DH = HIDDEN // HEADS
N_GRAPHS = 64
NPG = 64
EPG = 256
NEG_INF = -1e9
F32 = jnp.float32


def _im3(g):
    return (g, 0, 0)


def _im4(g):
    return (g, 0, 0, 0)


def _c2(g):
    return (0, 0)


def _gspec(shape):
    ndim = len(shape) + 1
    return pl.BlockSpec((1,) + shape, _im3 if ndim == 3 else _im4)


def _wspec(shape):
    return pl.BlockSpec(shape, lambda g: (0,) * len(shape))


def _dot(a, b):
    return jnp.dot(a, b, preferred_element_type=F32)


def _ln(h, g, b):
    m = jnp.mean(h, axis=-1, keepdims=True)
    d = h - m
    v = jnp.mean(d * d, axis=-1, keepdims=True)
    return d / jnp.sqrt(v + 1e-5) * g + b


def _p0_body(xg_ref, sc_ref, dc_ref, ea_ref, w1_ref, b1_ref, w2_ref, b2_ref,
             h_ref):
    xg = xg_ref[0]
    sc = sc_ref[0]
    dc = dc_ref[0]
    ea = ea_ref[0]
    iota = jax.lax.broadcasted_iota(jnp.int32, (EPG, NPG), 1)
    ohs = (sc == iota).astype(F32)
    ohd = (dc == iota).astype(F32)
    xs = jnp.dot(ohs, xg, preferred_element_type=F32,
                 precision=jax.lax.Precision.HIGHEST)
    xd = jnp.dot(ohd, xg, preferred_element_type=F32,
                 precision=jax.lax.Precision.HIGHEST)
    w1 = w1_ref[...]
    pre = ((_dot(xs, w1[:NODE_DIM]) + _dot(xd, w1[NODE_DIM:2 * NODE_DIM]))
           + _dot(ea, w1[2 * NODE_DIM:])) + b1_ref[...]
    h_ref[0] = _dot(jnp.maximum(pre, 0.0), w2_ref[...]) + b2_ref[...]


def _p1_body(h_ref, sc_ref, sr_ref, dc_ref, dr_ref,
             wq_ref, bq_ref, wk_ref, bk_ref, wv_ref, bv_ref,
             s_ref, v_ref, *, masked):
    h = h_ref[0]
    q = _dot(h, wq_ref[...]) + bq_ref[...]
    k = _dot(h, wk_ref[...]) + bk_ref[...]
    v = _dot(h, wv_ref[...]) + bv_ref[...]
    v_ref[0] = v
    if masked:
        sc = sc_ref[0]
        sr = sr_ref[0]
        dc = dc_ref[0]
        dr = dr_ref[0]
        adj = (sc == sr) | (sc == dr) | (dc == sr) | (dc == dr)
    for hh in range(HEADS):
        s = jax.lax.dot_general(q[:, hh * DH:(hh + 1) * DH],
                                k[:, hh * DH:(hh + 1) * DH],
                                (((1,), (1,)), ((), ())),
                                preferred_element_type=F32) / np.sqrt(DH)
        if masked:
            s = jnp.where(adj, s, NEG_INF)
        s_ref[0, hh] = s


def _p3_body(h1_ref, f1w_ref, f1b_ref, f2w_ref, f2b_ref, f_ref):
    h1 = h1_ref[0]
    t = jnp.maximum(_dot(h1, f1w_ref[...]) + f1b_ref[...], 0.0)
    f_ref[0] = _dot(t, f2w_ref[...]) + f2b_ref[...]


def _p4_body(h_ref, seed_ref, wq_ref, bq_ref, wk_ref, bk_ref, wv_ref, bv_ref,
             s_ref, v_ref):
    h = h_ref[0]
    qs = _dot(seed_ref[...], wq_ref[...]) + bq_ref[...]
    k = _dot(h, wk_ref[...]) + bk_ref[...]
    v = _dot(h, wv_ref[...]) + bv_ref[...]
    v_ref[0] = v
    for hh in range(HEADS):
        s = jax.lax.dot_general(qs[:, hh * DH:(hh + 1) * DH],
                                k[:, hh * DH:(hh + 1) * DH],
                                (((1,), (1,)), ((), ())),
                                preferred_element_type=F32) / np.sqrt(DH)
        s_ref[0, hh] = s


def _p6_body(h1_ref, f1w_ref, f1b_ref, f2w_ref, f2b_ref, f_ref):
    h1 = h1_ref[...]
    t = jnp.maximum(_dot(h1, f1w_ref[...]) + f1b_ref[...], 0.0)
    f_ref[...] = _dot(t, f2w_ref[...]) + f2b_ref[...]


def _p7_body(p_ref, ow1_ref, ob1_ref, ow2_ref, l_ref):
    t = jnp.maximum(_dot(p_ref[...], ow1_ref[...]) + ob1_ref[...], 0.0)
    l_ref[...] = _dot(t, ow2_ref[...])


def kernel(x, edge_index, edge_attr, batch, params):
    src = edge_index[0].astype(jnp.int32)
    dst = edge_index[1].astype(jnp.int32)
    base = (jnp.arange(N_GRAPHS, dtype=jnp.int32) * NPG)[:, None]
    src_l = src.reshape(N_GRAPHS, EPG) - base
    dst_l = dst.reshape(N_GRAPHS, EPG) - base
    sc = src_l.reshape(N_GRAPHS, EPG, 1)
    sr = src_l.reshape(N_GRAPHS, 1, EPG)
    dc = dst_l.reshape(N_GRAPHS, EPG, 1)
    dr = dst_l.reshape(N_GRAPHS, 1, EPG)
    x3 = x.reshape(N_GRAPHS, NPG, NODE_DIM)
    ea = edge_attr.reshape(N_GRAPHS, EPG, EDGE_DIM)

    p = params
    b1 = p['in_mlp']['b1'].reshape(1, HIDDEN)
    b2 = p['in_mlp']['b2'].reshape(1, HIDDEN)
    units = list(p['blocks']) + [p['pma']]
    seed = p['pma']['seed']
    ob1 = p['out_mlp']['b1'].reshape(1, HIDDEN)
    ow2 = jnp.pad(p['out_mlp']['W2'], ((0, 0), (0, 127)))

    grid = (N_GRAPHS,)
    hh_shape = jax.ShapeDtypeStruct((N_GRAPHS, EPG, HIDDEN), F32)

    h = pl.pallas_call(
        _p0_body, grid=grid,
        in_specs=[_gspec((NPG, NODE_DIM)), _gspec((EPG, 1)), _gspec((EPG, 1)),
                  _gspec((EPG, EDGE_DIM)),
                  _wspec((2 * NODE_DIM + EDGE_DIM, HIDDEN)), _wspec((1, HIDDEN)),
                  _wspec((HIDDEN, HIDDEN)), _wspec((1, HIDDEN))],
        out_specs=_gspec((EPG, HIDDEN)),
        out_shape=hh_shape,
    )(x3, sc, dc, ea, p['in_mlp']['W1'], b1, p['in_mlp']['W2'], b2)

    def softmax_av_ln(h_or_seed, s, v, u, seeded):
        def pg(args):
            if seeded:
                sg, vg = args
                qres = u['seed']
            else:
                hg, sg, vg = args
                qres = hg
            outs = []
            for hh in range(HEADS):
                sh = sg[hh]
                m = jnp.max(sh, axis=-1, keepdims=True)
                e = jnp.exp(sh - m)
                a = e / jnp.sum(e, axis=-1, keepdims=True)
                outs.append(jnp.dot(a, vg[:, hh * DH:(hh + 1) * DH],
                                    preferred_element_type=F32))
            o = jnp.concatenate(outs, axis=-1)
            att = jnp.dot(o, u['Wo'], preferred_element_type=F32) + u['bo']
            return _ln(qres + att, u['ln1g'], u['ln1b'])
        if seeded:
            return jax.vmap(lambda sg, vg: pg((sg, vg)))(s, v)
        return jax.vmap(lambda hg, sg, vg: pg((hg, sg, vg)))(h_or_seed, s, v)

    def attn_block(h, i, masked):
        u = units[i]
        bq = u['bq'].reshape(1, HIDDEN)
        bk = u['bk'].reshape(1, HIDDEN)
        bv = u['bv'].reshape(1, HIDDEN)
        s, v = pl.pallas_call(
            functools.partial(_p1_body, masked=masked), grid=grid,
            in_specs=[_gspec((EPG, HIDDEN)), _gspec((EPG, 1)), _gspec((1, EPG)),
                      _gspec((EPG, 1)), _gspec((1, EPG)),
                      _wspec((HIDDEN, HIDDEN)), _wspec((1, HIDDEN)),
                      _wspec((HIDDEN, HIDDEN)), _wspec((1, HIDDEN)),
                      _wspec((HIDDEN, HIDDEN)), _wspec((1, HIDDEN))],
            out_specs=[_gspec((HEADS, EPG, EPG)), _gspec((EPG, HIDDEN))],
            out_shape=[jax.ShapeDtypeStruct((N_GRAPHS, HEADS, EPG, EPG), F32),
                       hh_shape],
        )(h, sc, sr, dc, dr, u['Wq'], bq, u['Wk'], bk, u['Wv'], bv)
        h1 = softmax_av_ln(h, s, v, u, seeded=False)
        f = pl.pallas_call(
            _p3_body, grid=grid,
            in_specs=[_gspec((EPG, HIDDEN)),
                      _wspec((HIDDEN, HIDDEN)), _wspec((1, HIDDEN)),
                      _wspec((HIDDEN, HIDDEN)), _wspec((1, HIDDEN))],
            out_specs=_gspec((EPG, HIDDEN)),
            out_shape=hh_shape,
        )(h1, u['ff1W'], u['ff1b'].reshape(1, HIDDEN),
          u['ff2W'], u['ff2b'].reshape(1, HIDDEN))
        return jax.vmap(lambda h1g, fg: _ln(h1g + fg, u['ln2g'], u['ln2b']))(h1, f)

    for i in range(3):
        h = attn_block(h, i, masked=(i < 2))

    u = units[3]
    sp, vp = pl.pallas_call(
        _p4_body, grid=grid,
        in_specs=[_gspec((EPG, HIDDEN)), _wspec((1, HIDDEN)),
                  _wspec((HIDDEN, HIDDEN)), _wspec((1, HIDDEN)),
                  _wspec((HIDDEN, HIDDEN)), _wspec((1, HIDDEN)),
                  _wspec((HIDDEN, HIDDEN)), _wspec((1, HIDDEN))],
        out_specs=[_gspec((HEADS, 1, EPG)), _gspec((EPG, HIDDEN))],
        out_shape=[jax.ShapeDtypeStruct((N_GRAPHS, HEADS, 1, EPG), F32),
                   hh_shape],
    )(h, seed, u['Wq'], u['bq'].reshape(1, HIDDEN),
      u['Wk'], u['bk'].reshape(1, HIDDEN), u['Wv'], u['bv'].reshape(1, HIDDEN))
    h1p = softmax_av_ln(None, sp, vp, u, seeded=True)[:, 0, :]
    fp = pl.pallas_call(
        _p6_body,
        out_shape=jax.ShapeDtypeStruct((N_GRAPHS, HIDDEN), F32),
    )(h1p, u['ff1W'], u['ff1b'].reshape(1, HIDDEN),
      u['ff2W'], u['ff2b'].reshape(1, HIDDEN))
    pooled = jax.vmap(lambda h1g, fg: _ln(h1g + fg, u['ln2g'], u['ln2b']))(h1p, fp)

    logits = pl.pallas_call(
        _p7_body,
        out_shape=jax.ShapeDtypeStruct((N_GRAPHS, 128), F32),
    )(pooled, p['out_mlp']['W1'], ob1, ow2)
    return logits[:, 0] + p['out_mlp']['b2'][0]


# speedup vs baseline: 1.2401x; 1.2401x over previous
import jax
import jax.numpy as jnp
import numpy as np
from jax.experimental import pallas as pl

NODE_DIM = 128
EDGE_DIM = 16
HIDDEN = 256
HEADS = 8
DH = HIDDEN // HEADS
N_GRAPHS = 64
NPG = 64
EPG = 256
NEG_INF = -1e9


def _ln(h, g, b):
    m = jnp.mean(h, axis=-1, keepdims=True)
    d = h - m
    v = jnp.mean(d * d, axis=-1, keepdims=True)
    return d * jax.lax.rsqrt(v + 1e-5) * g + b


def _mha(q_in, kv_in, wqkv, bqkv, wo, bo, sbias):
    scale = 1.0 / np.sqrt(DH)
    qkv = jnp.dot(kv_in, wqkv, preferred_element_type=jnp.float32) + bqkv
    if q_in is kv_in:
        q = qkv[:, :HIDDEN] * scale
    else:
        q = (jnp.dot(q_in, wqkv[:, :HIDDEN], preferred_element_type=jnp.float32)
             + bqkv[:, :HIDDEN]) * scale
    k = qkv[:, HIDDEN:2 * HIDDEN]
    v = qkv[:, 2 * HIDDEN:]
    outs = []
    for hh in range(HEADS):
        qh = q[:, hh * DH:(hh + 1) * DH]
        kh = k[:, hh * DH:(hh + 1) * DH]
        vh = v[:, hh * DH:(hh + 1) * DH]
        s = jax.lax.dot_general(qh, kh, (((1,), (1,)), ((), ())),
                                preferred_element_type=jnp.float32)
        if sbias is not None:
            s = s + sbias
        m = jnp.max(s, axis=-1, keepdims=True)
        e = jnp.exp(s - m)
        r = 1.0 / jnp.sum(e, axis=-1, keepdims=True)
        oh = jnp.dot(e, vh, preferred_element_type=jnp.float32)
        outs.append(oh * r)
    o = jnp.concatenate(outs, axis=-1)
    return jnp.dot(o, wo, preferred_element_type=jnp.float32) + bo


def _ff(h1, f1w, f1b, f2w, f2b):
    t = jnp.maximum(jnp.dot(h1, f1w, preferred_element_type=jnp.float32) + f1b, 0.0)
    return jnp.dot(t, f2w, preferred_element_type=jnp.float32) + f2b


def _body(xg_ref, sc_ref, sr_ref, dc_ref, dr_ref, ea_ref,
          w1a_ref, w1b_ref, w1c_ref, b1_ref, w2_ref, b2_ref,
          wqkv_ref, wo_ref, f1w_ref, f2w_ref,
          bqkv_ref, bo_ref, f1b_ref, f2b_ref,
          l1g_ref, l1b_ref, l2g_ref, l2b_ref,
          seed_ref, ow1_ref, ob1_ref, ow2_ref, ob2_ref,
          out_ref):
    xg = xg_ref[0]
    sc = sc_ref[0]
    sr = sr_ref[0]
    dc = dc_ref[0]
    dr = dr_ref[0]
    ea = ea_ref[0]

    adj = (sc == sr) | (sc == dr) | (dc == sr) | (dc == dr)
    sbias = jnp.where(adj, 0.0, NEG_INF)

    iota = jax.lax.broadcasted_iota(jnp.int32, (EPG, NPG), 1)
    ohs = (sc == iota).astype(jnp.float32)
    ohd = (dc == iota).astype(jnp.float32)
    a_proj = jnp.dot(xg, w1a_ref[...], preferred_element_type=jnp.float32)
    b_proj = jnp.dot(xg, w1b_ref[...], preferred_element_type=jnp.float32)
    pre = (jnp.dot(ohs, a_proj, preferred_element_type=jnp.float32)
           + jnp.dot(ohd, b_proj, preferred_element_type=jnp.float32)
           + jnp.dot(ea, w1c_ref[...], preferred_element_type=jnp.float32)
           + b1_ref[...])
    h = jnp.dot(jnp.maximum(pre, 0.0), w2_ref[...],
                preferred_element_type=jnp.float32) + b2_ref[...]

    for i in range(3):
        att = _mha(h, h, wqkv_ref[i], bqkv_ref[i], wo_ref[i], bo_ref[i],
                   sbias if i < 2 else None)
        h1 = _ln(h + att, l1g_ref[i], l1b_ref[i])
        f = _ff(h1, f1w_ref[i], f1b_ref[i], f2w_ref[i], f2b_ref[i])
        h = _ln(h1 + f, l2g_ref[i], l2b_ref[i])

    s_seed = seed_ref[...]
    att = _mha(s_seed, h, wqkv_ref[3], bqkv_ref[3], wo_ref[3], bo_ref[3], None)
    h1 = _ln(s_seed + att, l1g_ref[3], l1b_ref[3])
    f = _ff(h1, f1w_ref[3], f1b_ref[3], f2w_ref[3], f2b_ref[3])
    pooled = _ln(h1 + f, l2g_ref[3], l2b_ref[3])

    t = jnp.maximum(jnp.dot(pooled, ow1_ref[...],
                            preferred_element_type=jnp.float32) + ob1_ref[...], 0.0)
    logit = jnp.sum(t * ow2_ref[...], axis=-1, keepdims=True)
    out_ref[0] = jnp.broadcast_to(logit, (1, 128)) + ob2_ref[...]


def _im3(g):
    return (g, 0, 0)


def _c2(g):
    return (0, 0)


def _c3(g):
    return (0, 0, 0)


def kernel(x, edge_index, edge_attr, batch, params):
    src = edge_index[0].astype(jnp.int32)
    dst = edge_index[1].astype(jnp.int32)
    base = (jnp.arange(N_GRAPHS, dtype=jnp.int32) * NPG)[:, None]
    src_l = src.reshape(N_GRAPHS, EPG) - base
    dst_l = dst.reshape(N_GRAPHS, EPG) - base
    sc = src_l.reshape(N_GRAPHS, EPG, 1)
    sr = src_l.reshape(N_GRAPHS, 1, EPG)
    dc = dst_l.reshape(N_GRAPHS, EPG, 1)
    dr = dst_l.reshape(N_GRAPHS, 1, EPG)
    x3 = x.reshape(N_GRAPHS, NPG, NODE_DIM)
    ea = jnp.pad(edge_attr, ((0, 0), (0, 128 - EDGE_DIM))).reshape(N_GRAPHS, EPG, 128)

    p = params
    w1 = p['in_mlp']['W1']
    w1a = w1[:NODE_DIM]
    w1b = w1[NODE_DIM:2 * NODE_DIM]
    w1c = jnp.pad(w1[2 * NODE_DIM:], ((0, 128 - EDGE_DIM), (0, 0)))
    b1 = p['in_mlp']['b1'].reshape(1, HIDDEN)
    w2 = p['in_mlp']['W2']
    b2 = p['in_mlp']['b2'].reshape(1, HIDDEN)

    units = list(p['blocks']) + [p['pma']]
    def stkw(name):
        return jnp.stack([u[name] for u in units])
    def stkb(name):
        return jnp.stack([u[name].reshape(1, HIDDEN) for u in units])
    WQKV = jnp.stack([jnp.concatenate([u['Wq'], u['Wk'], u['Wv']], axis=1)
                      for u in units])
    BQKV = jnp.stack([jnp.concatenate([u['bq'], u['bk'], u['bv']]).reshape(1, 3 * HIDDEN)
                      for u in units])
    WO = stkw('Wo')
    F1W, F2W = stkw('ff1W'), stkw('ff2W')
    BO = stkb('bo')
    F1B, F2B = stkb('ff1b'), stkb('ff2b')
    L1G, L1B = stkb('ln1g'), stkb('ln1b')
    L2G, L2B = stkb('ln2g'), stkb('ln2b')
    seed = p['pma']['seed'].reshape(1, HIDDEN)
    ow1 = p['out_mlp']['W1']
    ob1 = p['out_mlp']['b1'].reshape(1, HIDDEN)
    ow2 = p['out_mlp']['W2'].reshape(1, HIDDEN)
    ob2 = jnp.broadcast_to(p['out_mlp']['b2'].reshape(1, 1), (1, 128))

    wspec = [
        pl.BlockSpec((NODE_DIM, HIDDEN), _c2),
        pl.BlockSpec((NODE_DIM, HIDDEN), _c2),
        pl.BlockSpec((128, HIDDEN), _c2),
        pl.BlockSpec((1, HIDDEN), _c2),
        pl.BlockSpec((HIDDEN, HIDDEN), _c2),
        pl.BlockSpec((1, HIDDEN), _c2),
        pl.BlockSpec((4, HIDDEN, 3 * HIDDEN), _c3),
        pl.BlockSpec((4, HIDDEN, HIDDEN), _c3),
        pl.BlockSpec((4, HIDDEN, HIDDEN), _c3),
        pl.BlockSpec((4, HIDDEN, HIDDEN), _c3),
        pl.BlockSpec((4, 1, 3 * HIDDEN), _c3),
    ]
    wspec += [pl.BlockSpec((4, 1, HIDDEN), _c3)] * 7
    wspec += [
        pl.BlockSpec((1, HIDDEN), _c2),
        pl.BlockSpec((HIDDEN, HIDDEN), _c2),
        pl.BlockSpec((1, HIDDEN), _c2),
        pl.BlockSpec((1, HIDDEN), _c2),
        pl.BlockSpec((1, 128), _c2),
    ]

    out = pl.pallas_call(
        _body,
        grid=(N_GRAPHS,),
        in_specs=[
            pl.BlockSpec((1, NPG, NODE_DIM), _im3),
            pl.BlockSpec((1, EPG, 1), _im3),
            pl.BlockSpec((1, 1, EPG), _im3),
            pl.BlockSpec((1, EPG, 1), _im3),
            pl.BlockSpec((1, 1, EPG), _im3),
            pl.BlockSpec((1, EPG, 128), _im3),
        ] + wspec,
        out_specs=pl.BlockSpec((1, 1, 128), _im3),
        out_shape=jax.ShapeDtypeStruct((N_GRAPHS, 1, 128), jnp.float32),
    )(x3, sc, sr, dc, dr, ea,
      w1a, w1b, w1c, b1, w2, b2,
      WQKV, WO, F1W, F2W,
      BQKV, BO, F1B, F2B,
      L1G, L1B, L2G, L2B,
      seed, ow1, ob1, ow2, ob2)
    return out[:, 0, 0]
